# baseline (speedup 1.0000x reference)
import functools

import jax
import jax.numpy as jnp
from jax import lax
from jax.experimental import pallas as pl
from jax.experimental.pallas import tpu as pltpu
from jax.experimental.pallas import tpu_sc as plsc

_EPS = 1e-07
_B = 1024
_D = 128
_K = 1024
_N_DATA = 1000000
_T = 0.07

_NC = 2
_NS = 16
_L = 16
_NW = _NC * _NS
_RPW = _B // _NW
_CH = 64
_CPR = _K // _CH
_CPT = _RPW * _CPR
_J = _D // _L


def _norm_body(x_ref, o_ref):
    x = x_ref[...]
    o_ref[...] = x / jnp.sqrt(jnp.sum(x * x, axis=1, keepdims=True))


_normalize = pl.pallas_call(
    _norm_body, out_shape=jax.ShapeDtypeStruct((_B, _D), jnp.float32)
)


def _sc_body(mem_hbm, fsn_hbm, idx_hbm, cidx_hbm, en_hbm, ep_hbm,
             idxall, wbufs, ebuf, fsn_v, posbuf, pidx, pe,
             sems, psem):
    wid = lax.axis_index("s") * _NC + lax.axis_index("c")
    base = wid * _RPW

    pltpu.sync_copy(cidx_hbm.at[pl.ds(wid * (_CPT // 2), _CPT // 2)], idxall)
    pltpu.sync_copy(fsn_hbm.at[pl.ds(base, _RPW)], fsn_v)
    pltpu.sync_copy(idx_hbm.at[pl.ds(base, _RPW)], pidx)

    lanes = lax.iota(jnp.int32, _L)
    zeros = jnp.zeros((_L,), jnp.float32)

    def _row_dot(wref, r, fs):
        acc0 = wref[r, pl.ds(0, _L)] * fs[0]
        acc1 = wref[r, pl.ds(_L, _L)] * fs[1]
        for j in range(2, _J, 2):
            acc0 = acc0 + wref[r, pl.ds(j * _L, _L)] * fs[j]
            acc1 = acc1 + wref[r, pl.ds((j + 1) * _L, _L)] * fs[j + 1]
        return jnp.sum(acc0 + acc1)

    pcp = pltpu.async_copy(mem_hbm.at[pidx], posbuf, psem)

    _NB = len(wbufs)

    def _start(t, u):
        pltpu.async_copy(
            mem_hbm.at[idxall.at[t // 2, pl.ds(lax.rem(t, 2) * _CH, _CH)]],
            wbufs[u], sems[u],
        )

    def _wait(u):
        pltpu.make_async_copy(
            mem_hbm.at[pl.ds(0, _CH)], wbufs[u], sems[u]
        ).wait()

    def _compute(t, u):
        wb = wbufs[u]
        b = t // _CPR
        fs = [fsn_v[b, pl.ds(j * _L, _L)] for j in range(_J)]
        for g in range(_CH // _L):

            def nrow(r2, res):
                return jnp.where(
                    lanes == r2, _row_dot(wb, g * _L + r2, fs), res
                )

            ebuf[pl.ds((t % _CPR) * _CH + g * _L, _L)] = lax.fori_loop(
                0, _L, nrow, zeros, unroll=4
            )

    def _flush_row(t):
        b = t // _CPR

        def expg(g, carry):
            v = ebuf[pl.ds(g * _L, _L)]
            ebuf[pl.ds(g * _L, _L)] = jnp.exp(v / _T)
            return carry

        lax.fori_loop(0, _K // _L, expg, 0, unroll=4)
        pltpu.sync_copy(ebuf, en_hbm.at[pl.ds((base + b) * _K, _K)])

    for u in range(_NB - 1):
        _start(u, u)

    def step(s, carry):
        t0 = _NB * s
        for u in range(_NB):
            tt = t0 + u
            _wait(u)
            nxt = tt + _NB - 1

            @pl.when(nxt < _CPT)
            def _():
                _start(nxt, (u + _NB - 1) % _NB)

            _compute(tt, u)

            @pl.when(lax.rem(tt + 1, _CPR) == 0)
            def _():
                _flush_row(tt)

        return carry

    lax.fori_loop(0, _CPT // _NB, step, 0)

    pcp.wait()
    for g in range(_RPW // _L):

        def prow(r2, res):
            r = g * _L + r2
            fs = [fsn_v[r, pl.ds(j * _L, _L)] for j in range(_J)]
            return jnp.where(lanes == r2, _row_dot(posbuf, r, fs), res)

        pe[pl.ds(g * _L, _L)] = lax.fori_loop(0, _L, prow, zeros, unroll=4)

    def pexpg(g, carry):
        v = pe[pl.ds(g * _L, _L)]
        pe[pl.ds(g * _L, _L)] = jnp.exp(v / _T)
        return carry

    lax.fori_loop(0, _RPW // _L, pexpg, 0)

    pltpu.sync_copy(pe, ep_hbm.at[pl.ds(base, _RPW)])


_sc_scores = functools.partial(
    pl.kernel,
    out_type=[
        jax.ShapeDtypeStruct((_B * _K,), jnp.float32),
        jax.ShapeDtypeStruct((_B,), jnp.float32),
    ],
    mesh=plsc.VectorSubcoreMesh(core_axis_name="c", subcore_axis_name="s"),
    compiler_params=pltpu.CompilerParams(needs_layout_passes=False),
    scratch_types=[
        pltpu.VMEM((_CPT // 2, 2 * _CH), jnp.int32),
        [pltpu.VMEM((_CH, _D), jnp.float32) for _ in range(8)],
        pltpu.VMEM((_K,), jnp.float32),
        pltpu.VMEM((_RPW, _D), jnp.float32),
        pltpu.VMEM((_RPW, _D), jnp.float32),
        pltpu.VMEM((_RPW,), jnp.int32),
        pltpu.VMEM((_RPW,), jnp.float32),
        [pltpu.SemaphoreType.DMA for _ in range(8)],
        pltpu.SemaphoreType.DMA,
    ],
)(_sc_body)


def _loss_body(en_ref, ep_ref, o_ref):
    en = en_ref[...]
    ep = ep_ref[...]
    s = jnp.sum(en) + jnp.sum(ep)
    z = s / (_B * (_K + 1)) * _N_DATA
    c0 = _K * (1.0 / _N_DATA)
    pn = en / z
    pp = ep / z
    ld1 = jnp.sum(jnp.log(pp / (pp + c0 + _EPS)))
    ld0 = jnp.sum(jnp.log(c0 / (pn + c0 + _EPS)))
    o_ref[...] = (-(ld1 + ld0) / _B)[None, None]


_loss = pl.pallas_call(
    _loss_body, out_shape=jax.ShapeDtypeStruct((1, 1), jnp.float32)
)


def kernel(epoch, f_s, f_t, memory_t, idx, contrast_idx):
    fsn = _normalize(f_s.astype(jnp.float32))
    idx32 = idx.astype(jnp.int32)
    cidx = contrast_idx.astype(jnp.int32).reshape(_B * _K // (2 * _CH), 2 * _CH)
    en, ep = _sc_scores(memory_t, fsn, idx32, cidx)
    out = _loss(en.reshape(_B, _K), ep.reshape(_B // 128, 128))
    return out[0, 0]


# speedup vs baseline: 1.4605x; 1.4605x over previous
import functools

import jax
import jax.numpy as jnp
from jax import lax
from jax.experimental import pallas as pl
from jax.experimental.pallas import tpu as pltpu
from jax.experimental.pallas import tpu_sc as plsc

_EPS = 1e-07
_B = 1024
_D = 128
_K = 1024
_N_DATA = 1000000
_T = 0.07

_NC = 2
_NS = 16
_L = 16
_NW = _NC * _NS
_RPW = _B // _NW
_CH = 64
_CPR = _K // _CH
_CPT = _RPW * _CPR
_J = _D // _L


def _sc_body(mem_hbm, fsn_hbm, idx_hbm, cidx_hbm, en_hbm, ep_hbm,
             idxall, wbufs, ebuf, fsn_v, posbuf, pidx, pe,
             sems, psem):
    wid = lax.axis_index("s") * _NC + lax.axis_index("c")
    base = wid * _RPW

    pltpu.sync_copy(cidx_hbm.at[pl.ds(wid * _CPT, _CPT)], idxall)
    pltpu.sync_copy(fsn_hbm.at[pl.ds(base, _RPW)], fsn_v)
    pltpu.sync_copy(idx_hbm.at[pl.ds(base, _RPW)], pidx)

    lanes = lax.iota(jnp.int32, _L)
    zeros = jnp.zeros((_L,), jnp.float32)

    def _row_dot(wref, r, fs):
        acc0 = wref[r, pl.ds(0, _L)] * fs[0]
        acc1 = wref[r, pl.ds(_L, _L)] * fs[1]
        for j in range(2, _J, 2):
            acc0 = acc0 + wref[r, pl.ds(j * _L, _L)] * fs[j]
            acc1 = acc1 + wref[r, pl.ds((j + 1) * _L, _L)] * fs[j + 1]
        return jnp.sum(acc0 + acc1)

    for g in range(_RPW // _L):

        def nsq(r2, res):
            r = g * _L + r2
            fs = [fsn_v[r, pl.ds(j * _L, _L)] for j in range(_J)]
            return jnp.where(lanes == r2, _row_dot(fsn_v, r, fs), res)

        s2 = lax.fori_loop(0, _L, nsq, zeros, unroll=4)
        yi = jnp.int32(0x5F3759DF) - (plsc.bitcast(s2, jnp.int32) >> 1)
        y = plsc.bitcast(yi, jnp.float32)
        for _ in range(3):
            y = y * (1.5 - 0.5 * s2 * y * y)

        def scale_row(r2, carry):
            sc = jnp.sum(jnp.where(lanes == r2, y, 0.0))
            r = g * _L + r2
            for j in range(_J):
                fsn_v[r, pl.ds(j * _L, _L)] = fsn_v[r, pl.ds(j * _L, _L)] * sc
            return carry

        lax.fori_loop(0, _L, scale_row, 0)

    pcp = pltpu.async_copy(mem_hbm.at[pidx], posbuf, psem)

    _NB = len(wbufs)

    def _start(t, u):
        pltpu.async_copy(mem_hbm.at[idxall.at[t]], wbufs[u], sems[u])

    def _wait(u):
        pltpu.make_async_copy(
            mem_hbm.at[pl.ds(0, _CH)], wbufs[u], sems[u]
        ).wait()

    def _compute(t, u):
        wb = wbufs[u]
        b = t // _CPR
        fs = [fsn_v[b, pl.ds(j * _L, _L)] for j in range(_J)]
        for g in range(_CH // _L):

            def nrow(r2, res):
                return jnp.where(
                    lanes == r2, _row_dot(wb, g * _L + r2, fs), res
                )

            ebuf[pl.ds((t % _CPR) * _CH + g * _L, _L)] = lax.fori_loop(
                0, _L, nrow, zeros, unroll=4
            )

    def _flush_row(t):
        b = t // _CPR

        def expg(g, carry):
            v = ebuf[pl.ds(g * _L, _L)]
            ebuf[pl.ds(g * _L, _L)] = jnp.exp(v / _T)
            return carry

        lax.fori_loop(0, _K // _L, expg, 0, unroll=4)
        pltpu.sync_copy(ebuf, en_hbm.at[pl.ds((base + b) * _K, _K)])

    for u in range(_NB - 1):
        _start(u, u)

    def step(s, carry):
        t0 = _NB * s
        for u in range(_NB):
            tt = t0 + u
            _wait(u)
            nxt = tt + _NB - 1

            @pl.when(nxt < _CPT)
            def _():
                _start(nxt, (u + _NB - 1) % _NB)

            _compute(tt, u)

            @pl.when(lax.rem(tt + 1, _CPR) == 0)
            def _():
                _flush_row(tt)

        return carry

    lax.fori_loop(0, _CPT // _NB, step, 0)

    pcp.wait()
    for g in range(_RPW // _L):

        def prow(r2, res):
            r = g * _L + r2
            fs = [fsn_v[r, pl.ds(j * _L, _L)] for j in range(_J)]
            return jnp.where(lanes == r2, _row_dot(posbuf, r, fs), res)

        pe[pl.ds(g * _L, _L)] = lax.fori_loop(0, _L, prow, zeros, unroll=4)

    def pexpg(g, carry):
        v = pe[pl.ds(g * _L, _L)]
        pe[pl.ds(g * _L, _L)] = jnp.exp(v / _T)
        return carry

    lax.fori_loop(0, _RPW // _L, pexpg, 0)

    pltpu.sync_copy(pe, ep_hbm.at[pl.ds(base, _RPW)])


_sc_scores = functools.partial(
    pl.kernel,
    out_type=[
        jax.ShapeDtypeStruct((_B * _K,), jnp.float32),
        jax.ShapeDtypeStruct((_B,), jnp.float32),
    ],
    mesh=plsc.VectorSubcoreMesh(core_axis_name="c", subcore_axis_name="s"),
    compiler_params=pltpu.CompilerParams(needs_layout_passes=False),
    scratch_types=[
        pltpu.VMEM((_CPT, _CH), jnp.int32),
        [pltpu.VMEM((_CH, _D), jnp.float32) for _ in range(4)],
        pltpu.VMEM((_K,), jnp.float32),
        pltpu.VMEM((_RPW, _D), jnp.float32),
        pltpu.VMEM((_RPW, _D), jnp.float32),
        pltpu.VMEM((_RPW,), jnp.int32),
        pltpu.VMEM((_RPW,), jnp.float32),
        [pltpu.SemaphoreType.DMA for _ in range(4)],
        pltpu.SemaphoreType.DMA,
    ],
)(_sc_body)


def _loss_body(en_ref, ep_ref, o_ref):
    en = en_ref[...]
    ep = ep_ref[...]
    s = jnp.sum(en) + jnp.sum(ep)
    z = s / (_B * (_K + 1)) * _N_DATA
    c0 = _K * (1.0 / _N_DATA)
    pn = en / z
    pp = ep / z
    ld1 = jnp.sum(jnp.log(pp / (pp + c0 + _EPS)))
    ld0 = jnp.sum(jnp.log(c0 / (pn + c0 + _EPS)))
    o_ref[...] = (-(ld1 + ld0) / _B)[None, None]


_loss = pl.pallas_call(
    _loss_body, out_shape=jax.ShapeDtypeStruct((1, 1), jnp.float32)
)


def kernel(epoch, f_s, f_t, memory_t, idx, contrast_idx):
    idx32 = idx.astype(jnp.int32)
    cidx = contrast_idx.astype(jnp.int32).reshape(_B * _K // _CH, _CH)
    en, ep = _sc_scores(memory_t, f_s.astype(jnp.float32), idx32, cidx)
    out = _loss(en.reshape(_B, _K), ep.reshape(_B // 128, 128))
    return out[0, 0]
